# baseline (speedup 1.0000x reference)
import functools

import jax
import jax.numpy as jnp
from jax import lax
from jax.experimental import pallas as pl
from jax.experimental.pallas import tpu as pltpu
from jax.experimental.pallas import tpu_sc as plsc

_VB = 8192


def _sc_gather(emb_table, idx):
    n = idx.shape[0]
    d = emb_table.shape[1]
    npad = -(-n // 16) * 16
    idx_padded = jnp.pad(idx, (0, npad - n))
    mesh = plsc.VectorSubcoreMesh(core_axis_name="c", subcore_axis_name="s")

    @functools.partial(
        pl.kernel,
        out_type=jax.ShapeDtypeStruct((n, d), jnp.float32),
        mesh=mesh,
        scratch_types=[
            pltpu.VMEM((npad,), jnp.int32),
            pltpu.VMEM((n, d), jnp.float32),
            pltpu.SemaphoreType.DMA,
        ],
    )
    def gather_kernel(table_hbm, idx_hbm, out_hbm, idx_v, rows_v, sem):
        @pl.when((lax.axis_index("c") == 0) & (lax.axis_index("s") == 0))
        def _():
            pltpu.sync_copy(idx_hbm, idx_v)
            vecs = [idx_v[pl.ds(16 * g, 16)] for g in range(npad // 16)]
            copies = []
            for j in range(n):
                r = vecs[j // 16][j % 16]
                c = pltpu.make_async_copy(table_hbm.at[pl.ds(r, 1), :],
                                          rows_v.at[pl.ds(j, 1), :], sem)
                c.start()
                copies.append(c)
            for c in copies:
                c.wait()
            pltpu.sync_copy(rows_v, out_hbm)

    return gather_kernel(emb_table, idx_padded)


def _sc_stream_probe(W2):
    mesh = plsc.VectorSubcoreMesh(core_axis_name="c", subcore_axis_name="s")
    rows_per_w = 1024
    chunk = 64
    nchunks = rows_per_w // chunk

    @functools.partial(
        pl.kernel,
        out_type=jax.ShapeDtypeStruct((32, 16), jnp.float32),
        mesh=mesh,
        scratch_types=[
            pltpu.VMEM((2, chunk, 512), jnp.float32),
            pltpu.SemaphoreType.DMA((2,)),
        ],
    )
    def stream_kernel(w2_hbm, out_hbm, bufs, sems):
        w = lax.axis_index("c") * 16 + lax.axis_index("s")
        base = w * rows_per_w

        def start(i, slot):
            pltpu.make_async_copy(
                w2_hbm.at[pl.ds(base + i * chunk, chunk), :],
                bufs.at[slot], sems.at[slot]).start()

        def wait(slot):
            pltpu.make_async_copy(
                w2_hbm.at[pl.ds(base, chunk), :],
                bufs.at[slot], sems.at[slot]).wait()

        start(0, 0)
        start(1, 1)

        def body(i, carry):
            slot = lax.rem(i, 2)
            wait(slot)

            @pl.when(i + 2 < nchunks)
            def _():
                start(i + 2, slot)

            return carry

        lax.fori_loop(0, nchunks, body, 0)
        pltpu.sync_copy(bufs.at[0, 0, pl.ds(0, 16)], out_hbm.at[w])

    return stream_kernel(W2)


def _mlp_body(nb, vocab, emb_ref, w1_ref, b1_ref, w2_ref, b2_ref, out_ref,
              hid_ref, m_ref, s_ref):
    i = pl.program_id(0)

    @pl.when(i == 0)
    def _():
        h = lax.dot_general(emb_ref[...], w1_ref[...],
                            (((1,), (1,)), ((), ())),
                            preferred_element_type=jnp.float32)
        hid_ref[...] = jnp.maximum(h + b1_ref[...], 0.0)
        m_ref[...] = jnp.full((1, 1), -jnp.inf, jnp.float32)
        s_ref[...] = jnp.zeros((1, 1), jnp.float32)

    logits = lax.dot_general(hid_ref[...], w2_ref[...],
                             (((1,), (1,)), ((), ())),
                             preferred_element_type=jnp.float32)
    logits = logits + b2_ref[...]
    col = i * _VB + lax.broadcasted_iota(jnp.int32, (1, _VB), 1)
    logits = jnp.where(col < vocab, logits, -jnp.inf)
    out_ref[:, pl.ds(i * _VB, _VB)] = logits

    m_old = m_ref[...]
    m_new = jnp.maximum(m_old, jnp.max(logits, axis=(0, 1), keepdims=True))
    s_ref[...] = (s_ref[...] * jnp.exp(m_old - m_new)
                  + jnp.sum(jnp.exp(logits - m_new), axis=(0, 1), keepdims=True))
    m_ref[...] = m_new

    @pl.when(i == nb - 1)
    def _():
        lse = m_ref[...] + jnp.log(s_ref[...])
        out_ref[...] = out_ref[...] - lse[0, 0]


def kernel(inputs, emb_table, W1, b1, W2, b2):
    vocab, embd = emb_table.shape
    hid = W1.shape[0]
    nctx = inputs.shape[0]

    rows = _sc_gather(emb_table, inputs.astype(jnp.int32))
    emb = rows.reshape(1, nctx * embd)

    nb = -(-vocab // _VB)
    vpad = nb * _VB

    out = pl.pallas_call(
        functools.partial(_mlp_body, nb, vocab),
        grid=(nb,),
        in_specs=[
            pl.BlockSpec((1, nctx * embd), lambda i: (0, 0)),
            pl.BlockSpec((hid, nctx * embd), lambda i: (0, 0)),
            pl.BlockSpec((1, hid), lambda i: (0, 0)),
            pl.BlockSpec((_VB, hid), lambda i: (i, 0)),
            pl.BlockSpec((1, _VB), lambda i: (0, i)),
        ],
        out_specs=pl.BlockSpec((1, vpad), lambda i: (0, 0)),
        out_shape=jax.ShapeDtypeStruct((1, vpad), jnp.float32),
        scratch_shapes=[
            pltpu.VMEM((1, hid), jnp.float32),
            pltpu.VMEM((1, 1), jnp.float32),
            pltpu.VMEM((1, 1), jnp.float32),
        ],
    )(emb, W1, b1.reshape(1, hid), W2, b2.reshape(1, vocab))

    probe = _sc_stream_probe(W2)
    return out[:, :vocab] + probe[0, 0] * 0.0


# speedup vs baseline: 1.8769x; 1.8769x over previous
import functools

import jax
import jax.numpy as jnp
from jax import lax
from jax.experimental import pallas as pl
from jax.experimental.pallas import tpu as pltpu

_VB = 8192


def _gather_hid_body(nctx, s_ref, *refs):
    blk_refs = refs[:nctx]
    colx_ref, w1_ref, b1_ref, hid_ref = refs[nctx:]
    ext = jnp.concatenate([r[...] for r in blk_refs], axis=0)
    lanes = lax.broadcasted_iota(jnp.int32, ext.shape, 1)
    e_col = jnp.sum(jnp.where(lanes == colx_ref[...], ext, 0.0),
                    axis=1, keepdims=True)
    h_col = lax.dot_general(w1_ref[...], e_col, (((1,), (0,)), ((), ())),
                            preferred_element_type=jnp.float32)
    hid_ref[...] = jnp.maximum(h_col.T + b1_ref[...], 0.0)


def _tc_gather_hid(table_t, blkidx, colx, W1, b1):
    embd = table_t.shape[0]
    hid_dim = W1.shape[0]
    nctx = blkidx.shape[0]

    def make_map(j):
        return lambda i, s: (0, s[j])

    grid_spec = pltpu.PrefetchScalarGridSpec(
        num_scalar_prefetch=1,
        grid=(1,),
        in_specs=(
            [pl.BlockSpec((embd, 128), make_map(j)) for j in range(nctx)]
            + [
                pl.BlockSpec((nctx * embd, 1), lambda i, s: (0, 0)),
                pl.BlockSpec((hid_dim, nctx * embd), lambda i, s: (0, 0)),
                pl.BlockSpec((1, hid_dim), lambda i, s: (0, 0)),
            ]
        ),
        out_specs=pl.BlockSpec((1, hid_dim), lambda i, s: (0, 0)),
    )
    return pl.pallas_call(
        functools.partial(_gather_hid_body, nctx),
        grid_spec=grid_spec,
        out_shape=jax.ShapeDtypeStruct((1, hid_dim), jnp.float32),
    )(blkidx, *([table_t] * nctx), colx, W1, b1.reshape(1, hid_dim))


def _mlp_body(nb, vocab, hid_ref, w2_ref, b2_ref, out_ref, m_ref, s_ref):
    i = pl.program_id(0)

    @pl.when(i == 0)
    def _():
        m_ref[...] = jnp.full((1, 1), -jnp.inf, jnp.float32)
        s_ref[...] = jnp.zeros((1, 1), jnp.float32)

    logits = lax.dot_general(hid_ref[...], w2_ref[...],
                             (((1,), (1,)), ((), ())),
                             preferred_element_type=jnp.float32)
    logits = logits + b2_ref[...]
    col = i * _VB + lax.broadcasted_iota(jnp.int32, (1, _VB), 1)
    logits = jnp.where(col < vocab, logits, -jnp.inf)
    out_ref[:, pl.ds(i * _VB, _VB)] = logits

    m_old = m_ref[...]
    m_new = jnp.maximum(m_old, jnp.max(logits, axis=(0, 1), keepdims=True))
    s_ref[...] = (s_ref[...] * jnp.exp(m_old - m_new)
                  + jnp.sum(jnp.exp(logits - m_new), axis=(0, 1), keepdims=True))
    m_ref[...] = m_new

    @pl.when(i == nb - 1)
    def _():
        lse = m_ref[...] + jnp.log(s_ref[...])
        out_ref[...] = out_ref[...] - lse[0, 0]


def kernel(inputs, emb_table, W1, b1, W2, b2):
    vocab, embd = emb_table.shape
    hid_dim = W1.shape[0]
    nctx = inputs.shape[0]

    idx = inputs.astype(jnp.int32)
    colx = jnp.repeat(idx % 128, embd).reshape(nctx * embd, 1)
    hid = _tc_gather_hid(emb_table.T, idx // 128, colx, W1, b1)

    nb = -(-vocab // _VB)
    vpad = nb * _VB

    out = pl.pallas_call(
        functools.partial(_mlp_body, nb, vocab),
        grid=(nb,),
        in_specs=[
            pl.BlockSpec((1, hid_dim), lambda i: (0, 0)),
            pl.BlockSpec((_VB, hid_dim), lambda i: (i, 0)),
            pl.BlockSpec((1, _VB), lambda i: (0, i)),
        ],
        out_specs=pl.BlockSpec((1, vpad), lambda i: (0, 0)),
        out_shape=jax.ShapeDtypeStruct((1, vpad), jnp.float32),
        scratch_shapes=[
            pltpu.VMEM((1, 1), jnp.float32),
            pltpu.VMEM((1, 1), jnp.float32),
        ],
    )(hid, W2, b2.reshape(1, vocab))

    return out[:, :vocab]
